# baseline (speedup 1.0000x reference)
import functools

import jax
import jax.numpy as jnp
from jax import lax
from jax.experimental import pallas as pl
from jax.experimental.pallas import tpu as pltpu
from jax.experimental.pallas import tpu_sc as plsc

_NC = 2
_NS = 16
_NW = _NC * _NS
_LANES = 16

_S = 10

_SC_PARAMS = pltpu.CompilerParams(use_tc_tiling_on_sc=False,
                                  needs_layout_passes=False)


def _widx():
    return lax.axis_index("s") * _NC + lax.axis_index("c")


def _sc_sage(features, all_neighbors, nodes):
    n_rows, d = features.shape
    deg = all_neighbors.shape[1]
    b = nodes.shape[0]
    sb = b // _NW
    c = 32
    per_w = sb * (_S + 1)
    nch = per_w // c
    nself = sb // c
    assert per_w % c == 0 and nch % 2 == 0 and sb % _LANES == 0
    nv = d // _LANES
    ng = c // _LANES

    mesh = plsc.VectorSubcoreMesh(core_axis_name="c", subcore_axis_name="s")

    jb_types = [pltpu.VMEM((c, d), jnp.float32) for _ in range(2 * _S)]

    @functools.partial(
        pl.kernel,
        mesh=mesh,
        compiler_params=_SC_PARAMS,
        out_type=(jax.ShapeDtypeStruct((b * (_S + 1), d), jnp.float32),
                  jax.ShapeDtypeStruct((b * (_S + 1), d), jnp.float32)),
        scratch_types=[
            pltpu.VMEM((per_w,), jnp.int32),
            pltpu.VMEM((sb, deg), jnp.int32),
            pltpu.VMEM((c, deg), jnp.int32),
            pltpu.VMEM((c, deg), jnp.int32),
            pltpu.VMEM((_S, c), jnp.int32),
            pltpu.VMEM((_S, c), jnp.int32),
            pltpu.VMEM((c, d), jnp.float32),
            pltpu.VMEM((c, d), jnp.float32),
            pltpu.VMEM((c, d), jnp.float32),
            pltpu.VMEM((c, d), jnp.float32),
        ] + jb_types + [
            pltpu.SemaphoreType.DMA,
            pltpu.SemaphoreType.DMA,
            pltpu.SemaphoreType.DMA,
            pltpu.SemaphoreType.DMA,
            pltpu.SemaphoreType.DMA,
            pltpu.SemaphoreType.DMA,
            pltpu.SemaphoreType.DMA,
            pltpu.SemaphoreType.DMA,
            pltpu.SemaphoreType.DMA,
            pltpu.SemaphoreType.DMA,
        ],
    )
    def k(feat_hbm, an_hbm, nodes_hbm, self_hbm, nsum_hbm, ids_v, nbh,
          nbr0, nbr1, idx0, idx1, selfb0, selfb1, acc0, acc1, *rest):
        jbs = rest[:2 * _S]
        (sem_nbr0, sem_nbr1, sem_self0, sem_self1, sem_bank0, sem_bank1,
         sem_wS0, sem_wS1, sem_wA0, sem_wA1) = rest[2 * _S:]
        bank = (jbs[:_S], jbs[_S:])
        nbr = (nbr0, nbr1)
        idx = (idx0, idx1)
        selfb = (selfb0, selfb1)
        acc = (acc0, acc1)
        sem_nbr = (sem_nbr0, sem_nbr1)
        sem_self = (sem_self0, sem_self1)
        sem_bank = (sem_bank0, sem_bank1)
        sem_wS = (sem_wS0, sem_wS1)
        sem_wA = (sem_wA0, sem_wA1)

        w = _widx()
        nbase = w * sb * _S
        sbase = b * _S + w * sb

        rows16 = jnp.arange(_LANES, dtype=jnp.int32)

        def extract(src_ref, dst_ref, groups):
            for j in range(_S):
                cols = jnp.full((_LANES,), j, dtype=jnp.int32)
                for g in range(groups):
                    vals = plsc.load_gather(
                        src_ref, [rows16 + (g * _LANES), cols])
                    dst_ref[j, pl.ds(g * _LANES, _LANES)] = vals

        def fire_bank(ci_next, p_next):
            for j in range(_S):
                pltpu.async_copy(feat_hbm.at[idx[p_next].at[j]],
                                 bank[p_next][j], sem_bank[p_next])
            pltpu.async_copy(
                feat_hbm.at[ids_v.at[pl.ds(ci_next * c, c)]],
                selfb[p_next], sem_self[p_next])

        pltpu.sync_copy(nodes_hbm.at[pl.ds(w * sb, sb)],
                        ids_v.at[pl.ds(sb * _S, sb)])
        pltpu.async_copy(an_hbm.at[ids_v.at[pl.ds(sb * _S, sb)]],
                         nbh, sem_nbr0).wait()
        for j in range(_S):
            cols = jnp.full((_LANES,), j, dtype=jnp.int32)
            for g in range(sb // _LANES):
                srows = rows16 + (g * _LANES)
                vals = plsc.load_gather(nbh, [srows, cols])
                plsc.store_scatter(ids_v, [srows * _S + j], vals)

        pltpu.async_copy(an_hbm.at[ids_v.at[pl.ds(0, c)]],
                         nbr0, sem_nbr0).wait()
        extract(nbr0, idx0, ng)
        fire_bank(0, 0)
        pltpu.async_copy(an_hbm.at[ids_v.at[pl.ds(c, c)]], nbr1, sem_nbr1)

        def do_chunk(t, p):
            ci = 2 * t + p

            @pl.when(ci + 1 < nch)
            def _():
                pltpu.make_async_copy(an_hbm.at[ids_v.at[pl.ds(0, c)]],
                                      nbr[1 - p], sem_nbr[1 - p]).wait()
                extract(nbr[1 - p], idx[1 - p], ng)

            @pl.when(ci + 2 < nch)
            def _():
                pltpu.async_copy(
                    an_hbm.at[ids_v.at[pl.ds((ci + 2) * c, c)]],
                    nbr[p], sem_nbr[p])

            @pl.when(ci >= 1)
            def _():
                pltpu.make_async_copy(feat_hbm.at[pl.ds(0, c)],
                                      selfb[1 - p], sem_wS[1 - p]).wait()

            @pl.when(t >= 1)
            def _():
                pltpu.make_async_copy(feat_hbm.at[pl.ds(0, c)], acc[p],
                                      sem_wA[p]).wait()

            @pl.when(ci + 1 < nch)
            def _():
                fire_bank(ci + 1, 1 - p)

            for j in range(_S):
                pltpu.make_async_copy(feat_hbm.at[idx[p].at[j]],
                                      bank[p][j], sem_bank[p]).wait()

            def r_body(r, cr):
                for v in range(nv):
                    sl = pl.ds(v * _LANES, _LANES)
                    s = bank[p][0][r, sl]
                    for j in range(1, _S):
                        s = s + bank[p][j][r, sl]
                    acc[p][r, sl] = s
                return cr

            lax.fori_loop(0, c, r_body, 0)

            pltpu.make_async_copy(
                feat_hbm.at[ids_v.at[pl.ds(0, c)]], selfb[p],
                sem_self[p]).wait()
            orow = jnp.where(ci < nch - nself, nbase + ci * c,
                             sbase + (ci - (nch - nself)) * c)
            pltpu.async_copy(selfb[p], self_hbm.at[pl.ds(orow, c)], sem_wS[p])
            pltpu.async_copy(acc[p], nsum_hbm.at[pl.ds(orow, c)], sem_wA[p])

        def pair(t, carry):
            do_chunk(t, 0)
            do_chunk(t, 1)
            return carry

        lax.fori_loop(0, nch // 2, pair, 0)

        pltpu.make_async_copy(feat_hbm.at[pl.ds(0, c)], selfb[1],
                              sem_wS[1]).wait()
        for p in (0, 1):
            pltpu.make_async_copy(feat_hbm.at[pl.ds(0, c)], acc[p],
                                  sem_wA[p]).wait()

    return k(features, all_neighbors, nodes)


def _tc_head(selff, nsum, w1a, w1b, w2a, w2b, wc, b_seeds):
    nfeat = selff.shape[1]
    embed = w1a.shape[1]
    ncls = wc.shape[1]
    nb = 8
    bs = b_seeds // nb
    self_block0 = (b_seeds * _S) // bs

    def body(sn, nn, ss, ns, r1a, r1b, r2a, r2b, rc, o_ref):
        h1n = jnp.maximum(
            jnp.dot(sn[...], r1a[...], preferred_element_type=jnp.float32)
            + jnp.dot(nn[...], r1b[...], preferred_element_type=jnp.float32),
            0.0,
        )
        hsum = jnp.sum(h1n.reshape(bs, _S, embed), axis=1)
        h1s = jnp.maximum(
            jnp.dot(ss[...], r1a[...], preferred_element_type=jnp.float32)
            + jnp.dot(ns[...], r1b[...], preferred_element_type=jnp.float32),
            0.0,
        )
        emb = jnp.maximum(
            jnp.dot(h1s, r2a[...], preferred_element_type=jnp.float32)
            + jnp.dot(hsum, r2b[...], preferred_element_type=jnp.float32),
            0.0,
        )
        o_ref[...] = jax.nn.sigmoid(
            jnp.dot(emb, rc[...], preferred_element_type=jnp.float32)
        )

    wspec = lambda shp: pl.BlockSpec(shp, lambda i: (0, 0))
    return pl.pallas_call(
        body,
        grid=(nb,),
        in_specs=[
            pl.BlockSpec((bs * _S, nfeat), lambda i: (i, 0)),
            pl.BlockSpec((bs * _S, nfeat), lambda i: (i, 0)),
            pl.BlockSpec((bs, nfeat), lambda i: (i + self_block0, 0)),
            pl.BlockSpec((bs, nfeat), lambda i: (i + self_block0, 0)),
            wspec(w1a.shape),
            wspec(w1b.shape),
            wspec(w2a.shape),
            wspec(w2b.shape),
            wspec(wc.shape),
        ],
        out_specs=pl.BlockSpec((bs, ncls), lambda i: (i, 0)),
        out_shape=jax.ShapeDtypeStruct((b_seeds, ncls), jnp.float32),
    )(selff, nsum, selff, nsum, w1a, w1b, w2a, w2b, wc)


def kernel(nodes, all_neighbors, features, W1, W2, Wc):
    b = nodes.shape[0]
    nfeat = features.shape[1]
    embed = W1.shape[0]

    nodes = nodes.astype(jnp.int32)
    all_neighbors = all_neighbors.astype(jnp.int32)

    inv_s = jnp.float32(1.0 / _S)
    w1a = W1[:, :nfeat].T
    w1b = W1[:, nfeat:].T * inv_s
    w2a = W2[:, :embed].T
    w2b = W2[:, embed:].T * inv_s
    wct = Wc.T

    h = b // 2
    outs = []
    for lo in (0, h):
        selff, nsum = _sc_sage(features, all_neighbors,
                               lax.dynamic_slice_in_dim(nodes, lo, h))
        outs.append(_tc_head(selff, nsum, w1a, w1b, w2a, w2b, wct, h))
    return jnp.concatenate(outs, axis=0)


# speedup vs baseline: 1.0025x; 1.0025x over previous
import functools

import jax
import jax.numpy as jnp
from jax import lax
from jax.experimental import pallas as pl
from jax.experimental.pallas import tpu as pltpu
from jax.experimental.pallas import tpu_sc as plsc

_NC = 2
_NS = 16
_NW = _NC * _NS
_LANES = 16

_S = 10

_SC_PARAMS = pltpu.CompilerParams(use_tc_tiling_on_sc=False,
                                  needs_layout_passes=False)


def _widx():
    return lax.axis_index("s") * _NC + lax.axis_index("c")


def _sc_sage(features, all_neighbors, nodes):
    n_rows, d = features.shape
    deg = all_neighbors.shape[1]
    b = nodes.shape[0]
    sb = b // _NW
    c = 32
    per_w = sb * (_S + 1)
    nch = per_w // c
    nself = sb // c
    assert per_w % c == 0 and nch % 2 == 0 and sb % _LANES == 0
    nv = d // _LANES
    ng = c // _LANES

    mesh = plsc.VectorSubcoreMesh(core_axis_name="c", subcore_axis_name="s")

    jb_types = [pltpu.VMEM((c, d), jnp.float32) for _ in range(2 * _S)]

    @functools.partial(
        pl.kernel,
        mesh=mesh,
        compiler_params=_SC_PARAMS,
        out_type=(jax.ShapeDtypeStruct((b * (_S + 1), d), jnp.float32),
                  jax.ShapeDtypeStruct((b * (_S + 1), d), jnp.float32)),
        scratch_types=[
            pltpu.VMEM((per_w,), jnp.int32),
            pltpu.VMEM((sb, deg), jnp.int32),
            pltpu.VMEM((c, deg), jnp.int32),
            pltpu.VMEM((c, deg), jnp.int32),
            pltpu.VMEM((_S, c), jnp.int32),
            pltpu.VMEM((_S, c), jnp.int32),
            pltpu.VMEM((c, d), jnp.float32),
            pltpu.VMEM((c, d), jnp.float32),
            pltpu.VMEM((c, d), jnp.float32),
            pltpu.VMEM((c, d), jnp.float32),
        ] + jb_types + [
            pltpu.SemaphoreType.DMA,
            pltpu.SemaphoreType.DMA,
            pltpu.SemaphoreType.DMA,
            pltpu.SemaphoreType.DMA,
            pltpu.SemaphoreType.DMA,
            pltpu.SemaphoreType.DMA,
            pltpu.SemaphoreType.DMA,
            pltpu.SemaphoreType.DMA,
            pltpu.SemaphoreType.DMA,
            pltpu.SemaphoreType.DMA,
        ],
    )
    def k(feat_hbm, an_hbm, nodes_hbm, self_hbm, nsum_hbm, ids_v, nbh,
          nbr0, nbr1, idx0, idx1, selfb0, selfb1, acc0, acc1, *rest):
        jbs = rest[:2 * _S]
        (sem_nbr0, sem_nbr1, sem_self0, sem_self1, sem_bank0, sem_bank1,
         sem_wS0, sem_wS1, sem_wA0, sem_wA1) = rest[2 * _S:]
        bank = (jbs[:_S], jbs[_S:])
        nbr = (nbr0, nbr1)
        idx = (idx0, idx1)
        selfb = (selfb0, selfb1)
        acc = (acc0, acc1)
        sem_nbr = (sem_nbr0, sem_nbr1)
        sem_self = (sem_self0, sem_self1)
        sem_bank = (sem_bank0, sem_bank1)
        sem_wS = (sem_wS0, sem_wS1)
        sem_wA = (sem_wA0, sem_wA1)

        w = _widx()
        nbase = w * sb * _S
        sbase = b * _S + w * sb

        rows16 = jnp.arange(_LANES, dtype=jnp.int32)

        def extract(src_ref, dst_ref, groups):
            for j in range(_S):
                cols = jnp.full((_LANES,), j, dtype=jnp.int32)
                for g in range(groups):
                    vals = plsc.load_gather(
                        src_ref, [rows16 + (g * _LANES), cols])
                    dst_ref[j, pl.ds(g * _LANES, _LANES)] = vals

        def fire_bank(ci_next, p_next):
            for j in range(_S):
                pltpu.async_copy(feat_hbm.at[idx[p_next].at[j]],
                                 bank[p_next][j], sem_bank[p_next])
            pltpu.async_copy(
                feat_hbm.at[ids_v.at[pl.ds(ci_next * c, c)]],
                selfb[p_next], sem_self[p_next])

        pltpu.sync_copy(nodes_hbm.at[pl.ds(w * sb, sb)],
                        ids_v.at[pl.ds(sb * _S, sb)])
        pltpu.async_copy(an_hbm.at[ids_v.at[pl.ds(sb * _S, sb)]],
                         nbh, sem_nbr0).wait()
        for j in range(_S):
            cols = jnp.full((_LANES,), j, dtype=jnp.int32)
            for g in range(sb // _LANES):
                srows = rows16 + (g * _LANES)
                vals = plsc.load_gather(nbh, [srows, cols])
                plsc.store_scatter(ids_v, [srows * _S + j], vals)

        pltpu.async_copy(an_hbm.at[ids_v.at[pl.ds(0, c)]],
                         nbr0, sem_nbr0).wait()
        extract(nbr0, idx0, ng)
        fire_bank(0, 0)
        pltpu.async_copy(an_hbm.at[ids_v.at[pl.ds(c, c)]], nbr1, sem_nbr1)

        def do_chunk(t, p):
            ci = 2 * t + p

            @pl.when(ci + 1 < nch)
            def _():
                pltpu.make_async_copy(an_hbm.at[ids_v.at[pl.ds(0, c)]],
                                      nbr[1 - p], sem_nbr[1 - p]).wait()
                extract(nbr[1 - p], idx[1 - p], ng)

            @pl.when(ci + 2 < nch)
            def _():
                pltpu.async_copy(
                    an_hbm.at[ids_v.at[pl.ds((ci + 2) * c, c)]],
                    nbr[p], sem_nbr[p])

            @pl.when(ci >= 1)
            def _():
                pltpu.make_async_copy(feat_hbm.at[pl.ds(0, c)],
                                      selfb[1 - p], sem_wS[1 - p]).wait()

            @pl.when(t >= 1)
            def _():
                pltpu.make_async_copy(feat_hbm.at[pl.ds(0, c)], acc[p],
                                      sem_wA[p]).wait()

            @pl.when(ci + 1 < nch)
            def _():
                fire_bank(ci + 1, 1 - p)

            for j in range(_S):
                pltpu.make_async_copy(feat_hbm.at[idx[p].at[j]],
                                      bank[p][j], sem_bank[p]).wait()

            def r_body(r, cr):
                for v in range(nv):
                    sl = pl.ds(v * _LANES, _LANES)
                    s = bank[p][0][r, sl]
                    for j in range(1, _S):
                        s = s + bank[p][j][r, sl]
                    acc[p][r, sl] = s
                return cr

            lax.fori_loop(0, c, r_body, 0)

            pltpu.make_async_copy(
                feat_hbm.at[ids_v.at[pl.ds(0, c)]], selfb[p],
                sem_self[p]).wait()
            orow = jnp.where(ci < nch - nself, nbase + ci * c,
                             sbase + (ci - (nch - nself)) * c)
            pltpu.async_copy(selfb[p], self_hbm.at[pl.ds(orow, c)], sem_wS[p])
            pltpu.async_copy(acc[p], nsum_hbm.at[pl.ds(orow, c)], sem_wA[p])

        def pair(t, carry):
            do_chunk(t, 0)
            do_chunk(t, 1)
            return carry

        lax.fori_loop(0, nch // 2, pair, 0)

        pltpu.make_async_copy(feat_hbm.at[pl.ds(0, c)], selfb[1],
                              sem_wS[1]).wait()
        for p in (0, 1):
            pltpu.make_async_copy(feat_hbm.at[pl.ds(0, c)], acc[p],
                                  sem_wA[p]).wait()

    return k(features, all_neighbors, nodes)


def _tc_head(selff, nsum, w1a, w1b, w2a, w2b, wc, b_seeds):
    nfeat = selff.shape[1]
    embed = w1a.shape[1]
    ncls = wc.shape[1]
    nb = 16
    bs = b_seeds // nb
    self_block0 = (b_seeds * _S) // bs

    def body(sn, nn, ss, ns, r1a, r1b, r2a, r2b, rc, o_ref):
        h1n = jnp.maximum(
            jnp.dot(sn[...], r1a[...], preferred_element_type=jnp.float32)
            + jnp.dot(nn[...], r1b[...], preferred_element_type=jnp.float32),
            0.0,
        )
        hsum = jnp.sum(h1n.reshape(bs, _S, embed), axis=1)
        h1s = jnp.maximum(
            jnp.dot(ss[...], r1a[...], preferred_element_type=jnp.float32)
            + jnp.dot(ns[...], r1b[...], preferred_element_type=jnp.float32),
            0.0,
        )
        emb = jnp.maximum(
            jnp.dot(h1s, r2a[...], preferred_element_type=jnp.float32)
            + jnp.dot(hsum, r2b[...], preferred_element_type=jnp.float32),
            0.0,
        )
        o_ref[...] = jax.nn.sigmoid(
            jnp.dot(emb, rc[...], preferred_element_type=jnp.float32)
        )

    wspec = lambda shp: pl.BlockSpec(shp, lambda i: (0, 0))
    return pl.pallas_call(
        body,
        grid=(nb,),
        in_specs=[
            pl.BlockSpec((bs * _S, nfeat), lambda i: (i, 0)),
            pl.BlockSpec((bs * _S, nfeat), lambda i: (i, 0)),
            pl.BlockSpec((bs, nfeat), lambda i: (i + self_block0, 0)),
            pl.BlockSpec((bs, nfeat), lambda i: (i + self_block0, 0)),
            wspec(w1a.shape),
            wspec(w1b.shape),
            wspec(w2a.shape),
            wspec(w2b.shape),
            wspec(wc.shape),
        ],
        out_specs=pl.BlockSpec((bs, ncls), lambda i: (i, 0)),
        out_shape=jax.ShapeDtypeStruct((b_seeds, ncls), jnp.float32),
    )(selff, nsum, selff, nsum, w1a, w1b, w2a, w2b, wc)


def kernel(nodes, all_neighbors, features, W1, W2, Wc):
    b = nodes.shape[0]
    nfeat = features.shape[1]
    embed = W1.shape[0]

    nodes = nodes.astype(jnp.int32)
    all_neighbors = all_neighbors.astype(jnp.int32)

    selff, nsum = _sc_sage(features, all_neighbors, nodes)

    inv_s = jnp.float32(1.0 / _S)
    w1a = W1[:, :nfeat].T
    w1b = W1[:, nfeat:].T * inv_s
    w2a = W2[:, :embed].T
    w2b = W2[:, embed:].T * inv_s
    wct = Wc.T
    return _tc_head(selff, nsum, w1a, w1b, w2a, w2b, wct, b)


# speedup vs baseline: 1.0251x; 1.0225x over previous
import functools

import jax
import jax.numpy as jnp
from jax import lax
from jax.experimental import pallas as pl
from jax.experimental.pallas import tpu as pltpu
from jax.experimental.pallas import tpu_sc as plsc

_NC = 2
_NS = 16
_NW = _NC * _NS
_LANES = 16

_S = 10

_SC_PARAMS = pltpu.CompilerParams(use_tc_tiling_on_sc=False,
                                  needs_layout_passes=False)


def _widx():
    return lax.axis_index("s") * _NC + lax.axis_index("c")


def _sc_sage(features, all_neighbors, nodes):
    n_rows, d = features.shape
    deg = all_neighbors.shape[1]
    b = nodes.shape[0]
    sb = b // _NW
    c = 32
    per_w = sb * (_S + 1)
    nch = per_w // c
    nself = sb // c
    assert per_w % c == 0 and nch % 2 == 0 and sb % _LANES == 0
    nv = d // _LANES
    ng = c // _LANES

    mesh = plsc.VectorSubcoreMesh(core_axis_name="c", subcore_axis_name="s")

    jb_types = [pltpu.VMEM((c, d), jnp.float32) for _ in range(2 * _S)]

    @functools.partial(
        pl.kernel,
        mesh=mesh,
        compiler_params=_SC_PARAMS,
        out_type=(jax.ShapeDtypeStruct((b * (_S + 1), d), jnp.float32),
                  jax.ShapeDtypeStruct((b * (_S + 1), d), jnp.float32)),
        scratch_types=[
            pltpu.VMEM((per_w,), jnp.int32),
            pltpu.VMEM((sb, deg), jnp.int32),
            pltpu.VMEM((c, deg), jnp.int32),
            pltpu.VMEM((c, deg), jnp.int32),
            pltpu.VMEM((_S, c), jnp.int32),
            pltpu.VMEM((_S, c), jnp.int32),
            pltpu.VMEM((c, d), jnp.float32),
            pltpu.VMEM((c, d), jnp.float32),
            pltpu.VMEM((c, d), jnp.float32),
            pltpu.VMEM((c, d), jnp.float32),
        ] + jb_types + [
            pltpu.SemaphoreType.DMA,
            pltpu.SemaphoreType.DMA,
            pltpu.SemaphoreType.DMA,
            pltpu.SemaphoreType.DMA,
            pltpu.SemaphoreType.DMA,
            pltpu.SemaphoreType.DMA,
            pltpu.SemaphoreType.DMA,
            pltpu.SemaphoreType.DMA,
            pltpu.SemaphoreType.DMA,
            pltpu.SemaphoreType.DMA,
        ],
    )
    def k(feat_hbm, an_hbm, nodes_hbm, self_hbm, nsum_hbm, ids_v, nbh,
          nbr0, nbr1, idx0, idx1, selfb0, selfb1, acc0, acc1, *rest):
        jbs = rest[:2 * _S]
        (sem_nbr0, sem_nbr1, sem_self0, sem_self1, sem_bank0, sem_bank1,
         sem_wS0, sem_wS1, sem_wA0, sem_wA1) = rest[2 * _S:]
        bank = (jbs[:_S], jbs[_S:])
        nbr = (nbr0, nbr1)
        idx = (idx0, idx1)
        selfb = (selfb0, selfb1)
        acc = (acc0, acc1)
        sem_nbr = (sem_nbr0, sem_nbr1)
        sem_self = (sem_self0, sem_self1)
        sem_bank = (sem_bank0, sem_bank1)
        sem_wS = (sem_wS0, sem_wS1)
        sem_wA = (sem_wA0, sem_wA1)

        w = _widx()
        nbase = w * sb * _S
        sbase = b * _S + w * sb

        rows16 = jnp.arange(_LANES, dtype=jnp.int32)

        def extract(src_ref, dst_ref, groups):
            for j in range(_S):
                cols = jnp.full((_LANES,), j, dtype=jnp.int32)
                for g in range(groups):
                    vals = plsc.load_gather(
                        src_ref, [rows16 + (g * _LANES), cols])
                    dst_ref[j, pl.ds(g * _LANES, _LANES)] = vals

        def fire_bank(ci_next, p_next):
            for j in range(_S):
                pltpu.async_copy(feat_hbm.at[idx[p_next].at[j]],
                                 bank[p_next][j], sem_bank[p_next])
            pltpu.async_copy(
                feat_hbm.at[ids_v.at[pl.ds(ci_next * c, c)]],
                selfb[p_next], sem_self[p_next])

        pltpu.sync_copy(nodes_hbm.at[pl.ds(w * sb, sb)],
                        ids_v.at[pl.ds(sb * _S, sb)])
        pltpu.async_copy(an_hbm.at[ids_v.at[pl.ds(sb * _S, sb)]],
                         nbh, sem_nbr0).wait()
        for j in range(_S):
            cols = jnp.full((_LANES,), j, dtype=jnp.int32)
            for g in range(sb // _LANES):
                srows = rows16 + (g * _LANES)
                vals = plsc.load_gather(nbh, [srows, cols])
                plsc.store_scatter(ids_v, [srows * _S + j], vals)

        pltpu.async_copy(an_hbm.at[ids_v.at[pl.ds(0, c)]],
                         nbr0, sem_nbr0).wait()
        extract(nbr0, idx0, ng)
        fire_bank(0, 0)
        pltpu.async_copy(an_hbm.at[ids_v.at[pl.ds(c, c)]], nbr1, sem_nbr1)

        def do_chunk(t, p):
            ci = 2 * t + p

            @pl.when(ci + 1 < nch)
            def _():
                pltpu.make_async_copy(an_hbm.at[ids_v.at[pl.ds(0, c)]],
                                      nbr[1 - p], sem_nbr[1 - p]).wait()
                extract(nbr[1 - p], idx[1 - p], ng)

            @pl.when(ci + 2 < nch)
            def _():
                pltpu.async_copy(
                    an_hbm.at[ids_v.at[pl.ds((ci + 2) * c, c)]],
                    nbr[p], sem_nbr[p])

            @pl.when(ci >= 1)
            def _():
                pltpu.make_async_copy(feat_hbm.at[pl.ds(0, c)],
                                      selfb[1 - p], sem_wS[1 - p]).wait()

            @pl.when(t >= 1)
            def _():
                pltpu.make_async_copy(feat_hbm.at[pl.ds(0, c)], acc[p],
                                      sem_wA[p]).wait()

            @pl.when(ci + 1 < nch)
            def _():
                fire_bank(ci + 1, 1 - p)

            for j in range(_S):
                pltpu.make_async_copy(feat_hbm.at[idx[p].at[j]],
                                      bank[p][j], sem_bank[p]).wait()

            def r_body(r, cr):
                for v in range(nv):
                    sl = pl.ds(v * _LANES, _LANES)
                    s = bank[p][0][r, sl]
                    for j in range(1, _S):
                        s = s + bank[p][j][r, sl]
                    acc[p][r, sl] = s
                return cr

            lax.fori_loop(0, c, r_body, 0)

            pltpu.make_async_copy(
                feat_hbm.at[ids_v.at[pl.ds(0, c)]], selfb[p],
                sem_self[p]).wait()
            orow = jnp.where(ci < nch - nself, nbase + ci * c,
                             sbase + (ci - (nch - nself)) * c)
            pltpu.async_copy(selfb[p], self_hbm.at[pl.ds(orow, c)], sem_wS[p])
            pltpu.async_copy(acc[p], nsum_hbm.at[pl.ds(orow, c)], sem_wA[p])

        def pair(t, carry):
            do_chunk(t, 0)
            do_chunk(t, 1)
            return carry

        lax.fori_loop(0, nch // 2, pair, 0)

        pltpu.make_async_copy(feat_hbm.at[pl.ds(0, c)], selfb[1],
                              sem_wS[1]).wait()
        for p in (0, 1):
            pltpu.make_async_copy(feat_hbm.at[pl.ds(0, c)], acc[p],
                                  sem_wA[p]).wait()

    return k(features, all_neighbors, nodes)


def _tc_head(selff, nsum, w1a, w1b, w2a, w2b, wc, b_seeds):
    nfeat = selff.shape[1]
    embed = w1a.shape[1]
    ncls = wc.shape[1]
    nb = 8
    bs = b_seeds // nb
    self_block0 = (b_seeds * _S) // bs

    def body(sn, nn, ss, ns, r1a, r1b, r2a, r2b, rc, o_ref):
        h1n = jnp.maximum(
            jnp.dot(sn[...], r1a[...], preferred_element_type=jnp.float32)
            + jnp.dot(nn[...], r1b[...], preferred_element_type=jnp.float32),
            0.0,
        )
        hsum = jnp.sum(h1n.reshape(bs, _S, embed), axis=1)
        h1s = jnp.maximum(
            jnp.dot(ss[...], r1a[...], preferred_element_type=jnp.float32)
            + jnp.dot(ns[...], r1b[...], preferred_element_type=jnp.float32),
            0.0,
        )
        emb = jnp.maximum(
            jnp.dot(h1s, r2a[...], preferred_element_type=jnp.float32)
            + jnp.dot(hsum, r2b[...], preferred_element_type=jnp.float32),
            0.0,
        )
        o_ref[...] = jax.nn.sigmoid(
            jnp.dot(emb, rc[...], preferred_element_type=jnp.float32)
        )

    wspec = lambda shp: pl.BlockSpec(shp, lambda i: (0, 0))
    return pl.pallas_call(
        body,
        grid=(nb,),
        in_specs=[
            pl.BlockSpec((bs * _S, nfeat), lambda i: (i, 0)),
            pl.BlockSpec((bs * _S, nfeat), lambda i: (i, 0)),
            pl.BlockSpec((bs, nfeat), lambda i: (i + self_block0, 0)),
            pl.BlockSpec((bs, nfeat), lambda i: (i + self_block0, 0)),
            wspec(w1a.shape),
            wspec(w1b.shape),
            wspec(w2a.shape),
            wspec(w2b.shape),
            wspec(wc.shape),
        ],
        out_specs=pl.BlockSpec((bs, ncls), lambda i: (i, 0)),
        out_shape=jax.ShapeDtypeStruct((b_seeds, ncls), jnp.float32),
    )(selff, nsum, selff, nsum, w1a, w1b, w2a, w2b, wc)


def kernel(nodes, all_neighbors, features, W1, W2, Wc):
    b = nodes.shape[0]
    nfeat = features.shape[1]
    embed = W1.shape[0]

    nodes = nodes.astype(jnp.int32)
    all_neighbors = all_neighbors.astype(jnp.int32)

    selff, nsum = _sc_sage(features, all_neighbors, nodes)

    inv_s = jnp.float32(1.0 / _S)
    w1a = W1[:, :nfeat].T
    w1b = W1[:, nfeat:].T * inv_s
    w2a = W2[:, :embed].T
    w2b = W2[:, embed:].T * inv_s
    wct = Wc.T
    return _tc_head(selff, nsum, w1a, w1b, w2a, w2b, wct, b)
